# baseline (speedup 1.0000x reference)
import functools

import jax
import jax.numpy as jnp
from jax import lax
from jax.experimental import pallas as pl
from jax.experimental.pallas import tpu as pltpu
from jax.experimental.pallas import tpu_sc as plsc

NU = 5000
NN = 10000
NN2 = 10240
D = 256
DH = 128
NL = 4
NE = 160000
NC = 2
NS = 16
CH = 256
NCHT = NE // CH
RPT = NN2 // NS
MR = 128

_mesh = plsc.VectorSubcoreMesh(core_axis_name="c", subcore_axis_name="s")


def _f32(shape):
    return jax.ShapeDtypeStruct(shape, jnp.float32)


@functools.partial(
    pl.kernel,
    out_type=[_f32((2 * NN2, DH))] * 5,
    mesh=_mesh,
    scratch_types=[
        pltpu.VMEM((CH, DH), jnp.float32),
        pltpu.VMEM((CH,), jnp.int32),
        pltpu.VMEM((CH,), jnp.int32),
        pltpu.VMEM((CH,), jnp.float32),
        pltpu.VMEM_SHARED((NN2, DH), jnp.float32),
        pltpu.SemaphoreType.DMA,
    ],
)
def _lightgcn_sc(src_hbm, dst_hbm, w_hbm, x_hbm,
                 y_hbm, l1_hbm, l2_hbm, l3_hbm, l4_hbm,
                 rows_v, sidx_v, didx_v, w_v, acc_sh, sem):
    c = lax.axis_index("c")
    s = lax.axis_index("s")
    coff = c * NN2
    zv = jnp.zeros((16,), jnp.float32)

    def layer(src_tab, out_tab):
        def zrow(r, cy):
            for k in range(DH // 16):
                rows_v[r, pl.ds(k * 16, 16)] = zv
            return cy

        lax.fori_loop(0, MR, zrow, 0)
        for j in range(RPT // MR):
            pltpu.sync_copy(rows_v.at[pl.ds(0, MR)],
                            acc_sh.at[pl.ds(s * RPT + j * MR, MR)])
        plsc.subcore_barrier()

        nch = (NCHT - s + NS - 1) // NS

        def chunk(i, carry):
            base = (s + NS * i) * CH
            pltpu.sync_copy(src_hbm.at[pl.ds(base, CH)], sidx_v)
            pltpu.sync_copy(dst_hbm.at[pl.ds(base, CH)], didx_v)
            pltpu.sync_copy(w_hbm.at[pl.ds(base, CH)], w_v)

            def rebase(j, cy):
                sl = pl.ds(j * 16, 16)
                sidx_v[sl] = sidx_v[sl] + coff
                return cy

            lax.fori_loop(0, CH // 16, rebase, 0)

            pltpu.async_copy(src_tab.at[sidx_v], rows_v, sem).wait()

            def scale_grp(g, cy):
                wgrp = w_v[pl.ds(g * 16, 16)]
                for lane in range(16):
                    wb = wgrp.at[jnp.full((16,), lane, jnp.int32)].get(
                        mode="promise_in_bounds")
                    r = g * 16 + lane
                    for k in range(DH // 16):
                        sl = pl.ds(k * 16, 16)
                        rows_v[r, sl] = rows_v[r, sl] * wb
                return cy

            lax.fori_loop(0, CH // 16, scale_grp, 0)

            pltpu.sync_copy(rows_v, acc_sh.at[didx_v], add=True)
            return carry

        lax.fori_loop(0, nch, chunk, 0)
        plsc.subcore_barrier()

        pltpu.sync_copy(acc_sh.at[pl.ds(s * RPT, RPT)],
                        out_tab.at[pl.ds(coff + s * RPT, RPT)])
        plsc.subcore_barrier()

    layer(x_hbm, l1_hbm)
    layer(l1_hbm, l2_hbm)
    layer(l2_hbm, l3_hbm)
    layer(l3_hbm, l4_hbm)

    w = c * NS + s
    for j in range(RPT // MR):
        rbase = w * RPT + j * MR
        pltpu.sync_copy(x_hbm.at[pl.ds(rbase, MR)], rows_v.at[pl.ds(0, MR)])
        for tab in (l1_hbm, l2_hbm, l3_hbm, l4_hbm):
            pltpu.sync_copy(tab.at[pl.ds(rbase, MR)], rows_v.at[pl.ds(MR, MR)])

            def macc(r, cy):
                for k in range(DH // 16):
                    sl = pl.ds(k * 16, 16)
                    rows_v[r, sl] = rows_v[r, sl] + rows_v[MR + r, sl]
                return cy

            lax.fori_loop(0, MR, macc, 0)

        def mscale(r, cy):
            for k in range(DH // 16):
                sl = pl.ds(k * 16, 16)
                rows_v[r, sl] = rows_v[r, sl] * jnp.float32(1.0 / (NL + 1))
            return cy

        lax.fori_loop(0, MR, mscale, 0)
        pltpu.sync_copy(rows_v.at[pl.ds(0, MR)], y_hbm.at[pl.ds(rbase, MR)])


def kernel(edge_index, edge_weight, user_emb, item_emb):
    src = edge_index[0].astype(jnp.int32)
    dst = edge_index[1].astype(jnp.int32)
    all_emb = jnp.concatenate([user_emb, item_emb], axis=0)
    pad = jnp.zeros((NN2 - NN, DH), jnp.float32)
    x2 = jnp.concatenate(
        [all_emb[:, :DH], pad, all_emb[:, DH:], pad], axis=0)
    y = _lightgcn_sc(src, dst, edge_weight, x2)[0]
    final = jnp.concatenate([y[:NN], y[NN2:NN2 + NN]], axis=1)
    return (final[:NU], final[NU:])
